# Initial kernel scaffold; baseline (speedup 1.0000x reference)
import functools

import jax
import jax.numpy as jnp
from jax import lax
from jax.experimental import pallas as pl
from jax.experimental.pallas import tpu as pltpu
from jax.experimental.pallas import tpu_sc as plsc

N = 10000
NPAD = 10240
E = 320000
EMB = 64
NUM_GRAPHS = 16

C = 512
NCHUNK = E // C
FPT = 4
BN = 1024
GRID = NPAD // BN

_SC_MESH = plsc.VectorSubcoreMesh(core_axis_name="c", subcore_axis_name="s")


def _sc_aggregate(x_t, src, dst, attr, wv, bv):
    out_t = [jax.ShapeDtypeStruct((2, EMB, NPAD), jnp.float32),
             jax.ShapeDtypeStruct((2, EMB, NPAD), jnp.float32)]

    @functools.partial(
        pl.kernel,
        out_type=out_t,
        mesh=_SC_MESH,
        scratch_types=[
            pltpu.VMEM((FPT, NPAD), jnp.float32),
            pltpu.VMEM((FPT, NPAD), jnp.float32),
            pltpu.VMEM((FPT, NPAD), jnp.float32),
            pltpu.VMEM((C,), jnp.int32),
            pltpu.VMEM((C,), jnp.int32),
            pltpu.VMEM((C,), jnp.float32),
            pltpu.VMEM((EMB,), jnp.float32),
            pltpu.VMEM((EMB,), jnp.float32),
        ],
    )
    def k(x_hbm, src_hbm, dst_hbm, attr_hbm, w_hbm, b_hbm, outA, outB,
          xv, aggA, aggB, srcv, dstv, attrv, wvv, bvv):
        cid = lax.axis_index("c")
        sid = lax.axis_index("s")
        wid = cid * 16 + sid
        g = wid % 16
        half = wid // 16

        pltpu.sync_copy(x_hbm.at[pl.ds(g * FPT, FPT)], xv)
        pltpu.sync_copy(w_hbm, wvv)
        pltpu.sync_copy(b_hbm, bvv)

        zero = jnp.zeros((16,), jnp.float32)

        @pl.loop(0, NPAD // 16)
        def _(i):
            for f in range(FPT):
                aggA[f, pl.ds(i * 16, 16)] = zero
                aggB[f, pl.ds(i * 16, 16)] = zero

        wb = [plsc.load_gather(wvv, [jnp.full((16,), g * FPT + f, jnp.int32)])
              for f in range(FPT)]
        bb = [plsc.load_gather(bvv, [jnp.full((16,), g * FPT + f, jnp.int32)])
              for f in range(FPT)]

        @pl.loop(half, NCHUNK, step=2)
        def _(kc):
            base = kc * C
            pltpu.sync_copy(src_hbm.at[pl.ds(base, C)], srcv)
            pltpu.sync_copy(dst_hbm.at[pl.ds(base, C)], dstv)
            pltpu.sync_copy(attr_hbm.at[pl.ds(base, C)], attrv)

            @pl.loop(0, C // 16)
            def _(i):
                sl = pl.ds(i * 16, 16)
                sv = srcv[sl]
                dv = dstv[sl]
                av = attrv[sl]
                for f in range(FPT):
                    fidx = jnp.full((16,), f, jnp.int32)
                    ee = av * wb[f] + bb[f]
                    xa = plsc.load_gather(xv, [fidx, sv])
                    plsc.addupdate_scatter(aggA, [fidx, dv],
                                           jnp.maximum(xa + ee, 0.0))
                    xb = plsc.load_gather(xv, [fidx, dv])
                    plsc.addupdate_scatter(aggB, [fidx, sv],
                                           jnp.maximum(xb + ee, 0.0))

        pltpu.sync_copy(aggA, outA.at[half, pl.ds(g * FPT, FPT)])
        pltpu.sync_copy(aggB, outB.at[half, pl.ds(g * FPT, FPT)])

    return k(x_t, src, dst, attr, wv, bv)


def _mm(a, b):
    return lax.dot_general(a, b, (((1,), (0,)), ((), ())),
                           preferred_element_type=jnp.float32)


def _tc_encode(nodes_pad, W_enc, b_enc_col):
    d_in = nodes_pad.shape[1]

    def body(n_ref, w_ref, b_ref, o_ref):
        o_ref[...] = lax.dot_general(
            w_ref[...], n_ref[...], (((1,), (1,)), ((), ())),
            preferred_element_type=jnp.float32) + b_ref[...]

    return pl.pallas_call(
        body,
        grid=(GRID,),
        in_specs=[
            pl.BlockSpec((BN, d_in), lambda j: (j, 0)),
            pl.BlockSpec((EMB, d_in), lambda j: (0, 0)),
            pl.BlockSpec((EMB, 1), lambda j: (0, 0)),
        ],
        out_specs=pl.BlockSpec((EMB, BN), lambda j: (0, j)),
        out_shape=jax.ShapeDtypeStruct((EMB, NPAD), jnp.float32),
    )(nodes_pad, W_enc, b_enc_col)


def _tc_iter(x_t, pA, pB, W1a, b1a, W1b, b1b, W2a, b2a, W2b, b2b,
             Wf1, bf1, Wf2, bf2):
    def body(x_ref, pa_ref, pb_ref,
             w1a, b1a_, w1b, b1b_, w2a, b2a_, w2b, b2b_,
             wf1, bf1_, wf2, bf2_, o_ref):
        x = x_ref[...]
        hA = x + pa_ref[0] + pa_ref[1]
        hB = x + pb_ref[0] + pb_ref[1]
        fi = _mm(w1b[...], jnp.maximum(_mm(w1a[...], hA) + b1a_[...], 0.0)) \
            + b1b_[...]
        fo = _mm(w2b[...], jnp.maximum(_mm(w2a[...], hB) + b2a_[...], 0.0)) \
            + b2b_[...]
        cat = jnp.concatenate([x, fi, fo], axis=0)
        u = jnp.maximum(_mm(wf1[...], cat) + bf1_[...], 0.0)
        o_ref[...] = x + _mm(wf2[...], u) + bf2_[...]

    full = lambda shape: pl.BlockSpec(shape, lambda j: tuple(0 for _ in shape))
    return pl.pallas_call(
        body,
        grid=(GRID,),
        in_specs=[
            pl.BlockSpec((EMB, BN), lambda j: (0, j)),
            pl.BlockSpec((2, EMB, BN), lambda j: (0, 0, j)),
            pl.BlockSpec((2, EMB, BN), lambda j: (0, 0, j)),
            full((EMB, EMB)), full((EMB, 1)),
            full((EMB, EMB)), full((EMB, 1)),
            full((EMB, EMB)), full((EMB, 1)),
            full((EMB, EMB)), full((EMB, 1)),
            full((2 * EMB, 3 * EMB)), full((2 * EMB, 1)),
            full((EMB, 2 * EMB)), full((EMB, 1)),
        ],
        out_specs=pl.BlockSpec((EMB, BN), lambda j: (0, j)),
        out_shape=jax.ShapeDtypeStruct((EMB, NPAD), jnp.float32),
    )(x_t, pA, pB, W1a, b1a, W1b, b1b, W2a, b2a, W2b, b2b,
      Wf1, bf1, Wf2, bf2)


def _tc_final(x_t, Wc, bc_row, batch_row):
    DOUT = Wc.shape[0]

    def body(x_ref, wc_ref, bc_ref, bt_ref, o_ref, mx, sm, cn):
        j = pl.program_id(0)

        @pl.when(j == 0)
        def _():
            mx[...] = jnp.full((NUM_GRAPHS, DOUT), -jnp.inf, jnp.float32)
            sm[...] = jnp.zeros((NUM_GRAPHS, DOUT), jnp.float32)
            cn[...] = jnp.zeros((NUM_GRAPHS, 128), jnp.float32)

        yt = lax.dot_general(x_ref[...], wc_ref[...],
                             (((0,), (1,)), ((), ())),
                             preferred_element_type=jnp.float32) + bc_ref[...]
        bt = bt_ref[...]
        gids = lax.broadcasted_iota(jnp.int32, (NUM_GRAPHS, BN), 0)
        masks = (gids == bt).astype(jnp.float32)
        sm[...] += lax.dot_general(masks, yt, (((1,), (0,)), ((), ())),
                                   preferred_element_type=jnp.float32)
        cn[...] += jnp.sum(masks, axis=1, keepdims=True)
        for gph in range(NUM_GRAPHS):
            m = bt == gph
            ym = jnp.where(jnp.transpose(m), yt, -jnp.inf)
            gm = jnp.max(ym, axis=0, keepdims=True)
            mx[pl.ds(gph, 1), :] = jnp.maximum(mx[pl.ds(gph, 1), :], gm)

        @pl.when(j == GRID - 1)
        def _():
            o_ref[:, :DOUT] = mx[...]
            o_ref[:, DOUT:] = sm[...] / jnp.maximum(cn[:, :1], 1.0)

    return pl.pallas_call(
        body,
        grid=(GRID,),
        in_specs=[
            pl.BlockSpec((EMB, BN), lambda j: (0, j)),
            pl.BlockSpec((DOUT, EMB), lambda j: (0, 0)),
            pl.BlockSpec((1, DOUT), lambda j: (0, 0)),
            pl.BlockSpec((1, BN), lambda j: (0, j)),
        ],
        out_specs=pl.BlockSpec((NUM_GRAPHS, 2 * DOUT), lambda j: (0, 0)),
        out_shape=jax.ShapeDtypeStruct((NUM_GRAPHS, 2 * DOUT), jnp.float32),
        scratch_shapes=[
            pltpu.VMEM((NUM_GRAPHS, DOUT), jnp.float32),
            pltpu.VMEM((NUM_GRAPHS, DOUT), jnp.float32),
            pltpu.VMEM((NUM_GRAPHS, 128), jnp.float32),
        ],
    )(x_t, Wc, bc_row, batch_row)


def kernel(nodes, edges, edge_attr, batch,
           W_enc, b_enc, W_edge, b_edge,
           W1a, b1a, W1b, b1b, W2a, b2a, W2b, b2b,
           Wf1, bf1, Wf2, bf2, Wc, bc):
    src = edges[0]
    dst = edges[1]
    wv = W_edge[:, 0]
    bv = b_edge

    nodes_pad = jnp.pad(nodes, ((0, NPAD - N), (0, 0)))
    batch_row = jnp.pad(batch, (0, NPAD - N),
                        constant_values=NUM_GRAPHS).reshape(1, NPAD)

    x_t = _tc_encode(nodes_pad, W_enc, b_enc.reshape(EMB, 1))
    for _ in range(3):
        pA, pB = _sc_aggregate(x_t, src, dst, attr=edge_attr, wv=wv, bv=bv)
        x_t = _tc_iter(x_t, pA, pB,
                       W1a, b1a.reshape(EMB, 1), W1b, b1b.reshape(EMB, 1),
                       W2a, b2a.reshape(EMB, 1), W2b, b2b.reshape(EMB, 1),
                       Wf1, bf1.reshape(2 * EMB, 1),
                       Wf2, bf2.reshape(EMB, 1))
    return _tc_final(x_t, Wc, bc.reshape(1, 2 * EMB), batch_row)


# speedup vs baseline: 2.4939x; 2.4939x over previous
import dataclasses
import functools

import jax
import jax.numpy as jnp
from jax import lax
from jax.experimental import pallas as pl
from jax.experimental.pallas import tpu as pltpu
from jax.experimental.pallas import tpu_sc as plsc

N = 10000
NPAD = 10240
E = 320000
EMB = 64
NUM_GRAPHS = 16

C = 512
NCHUNK = E // C
FPT = 4
BN = 1024
GRID = NPAD // BN

_SC_MESH = plsc.VectorSubcoreMesh(core_axis_name="c", subcore_axis_name="s")

_SC_PARAMS = pltpu.CompilerParams()
if "needs_layout_passes" in pltpu.CompilerParams.__dataclass_fields__:
    _SC_PARAMS = dataclasses.replace(_SC_PARAMS, needs_layout_passes=False)


def _sc_aggregate(x_t, src, dst, attr, wv, bv):
    out_t = [jax.ShapeDtypeStruct((2, EMB, NPAD), jnp.float32),
             jax.ShapeDtypeStruct((2, EMB, NPAD), jnp.float32)]

    @functools.partial(
        pl.kernel,
        out_type=out_t,
        mesh=_SC_MESH,
        compiler_params=_SC_PARAMS,
        scratch_types=[
            pltpu.VMEM((FPT, NPAD), jnp.float32),
            pltpu.VMEM((FPT, NPAD), jnp.float32),
            pltpu.VMEM((FPT, NPAD), jnp.float32),
            pltpu.VMEM((C,), jnp.int32),
            pltpu.VMEM((C,), jnp.int32),
            pltpu.VMEM((C,), jnp.float32),
            pltpu.VMEM((EMB,), jnp.float32),
            pltpu.VMEM((EMB,), jnp.float32),
        ],
    )
    def k(x_hbm, src_hbm, dst_hbm, attr_hbm, w_hbm, b_hbm, outA, outB,
          xv, aggA, aggB, srcv, dstv, attrv, wvv, bvv):
        cid = lax.axis_index("c")
        sid = lax.axis_index("s")
        wid = cid * 16 + sid
        g = wid % 16
        half = wid // 16

        pltpu.sync_copy(x_hbm.at[pl.ds(g * FPT, FPT)], xv)
        pltpu.sync_copy(w_hbm, wvv)
        pltpu.sync_copy(b_hbm, bvv)

        zero = jnp.zeros((16,), jnp.float32)

        @pl.loop(0, NPAD // 16)
        def _(i):
            for f in range(FPT):
                aggA[f, pl.ds(i * 16, 16)] = zero
                aggB[f, pl.ds(i * 16, 16)] = zero

        wb = [plsc.load_gather(wvv, [jnp.full((16,), g * FPT + f, jnp.int32)])
              for f in range(FPT)]
        bb = [plsc.load_gather(bvv, [jnp.full((16,), g * FPT + f, jnp.int32)])
              for f in range(FPT)]

        @pl.loop(half, NCHUNK, step=2)
        def _(kc):
            base = kc * C
            pltpu.sync_copy(src_hbm.at[pl.ds(base, C)], srcv)
            pltpu.sync_copy(dst_hbm.at[pl.ds(base, C)], dstv)
            pltpu.sync_copy(attr_hbm.at[pl.ds(base, C)], attrv)

            @pl.loop(0, C // 16)
            def _(i):
                sl = pl.ds(i * 16, 16)
                sv = srcv[sl]
                dv = dstv[sl]
                av = attrv[sl]
                for f in range(FPT):
                    fidx = jnp.full((16,), f, jnp.int32)
                    ee = av * wb[f] + bb[f]
                    xa = plsc.load_gather(xv, [fidx, sv])
                    plsc.addupdate_scatter(aggA, [fidx, dv],
                                           jnp.maximum(xa + ee, 0.0))
                    xb = plsc.load_gather(xv, [fidx, dv])
                    plsc.addupdate_scatter(aggB, [fidx, sv],
                                           jnp.maximum(xb + ee, 0.0))

        pltpu.sync_copy(aggA, outA.at[half, pl.ds(g * FPT, FPT)])
        pltpu.sync_copy(aggB, outB.at[half, pl.ds(g * FPT, FPT)])

    return k(x_t, src, dst, attr, wv, bv)


def _mm(a, b):
    return lax.dot_general(a, b, (((1,), (0,)), ((), ())),
                           preferred_element_type=jnp.float32)


def _tc_encode(nodes_pad, W_enc, b_enc_col):
    d_in = nodes_pad.shape[1]

    def body(n_ref, w_ref, b_ref, o_ref):
        o_ref[...] = lax.dot_general(
            w_ref[...], n_ref[...], (((1,), (1,)), ((), ())),
            preferred_element_type=jnp.float32) + b_ref[...]

    return pl.pallas_call(
        body,
        grid=(GRID,),
        in_specs=[
            pl.BlockSpec((BN, d_in), lambda j: (j, 0)),
            pl.BlockSpec((EMB, d_in), lambda j: (0, 0)),
            pl.BlockSpec((EMB, 1), lambda j: (0, 0)),
        ],
        out_specs=pl.BlockSpec((EMB, BN), lambda j: (0, j)),
        out_shape=jax.ShapeDtypeStruct((EMB, NPAD), jnp.float32),
    )(nodes_pad, W_enc, b_enc_col)


def _tc_iter(x_t, pA, pB, W1a, b1a, W1b, b1b, W2a, b2a, W2b, b2b,
             Wf1, bf1, Wf2, bf2):
    def body(x_ref, pa_ref, pb_ref,
             w1a, b1a_, w1b, b1b_, w2a, b2a_, w2b, b2b_,
             wf1, bf1_, wf2, bf2_, o_ref):
        x = x_ref[...]
        hA = x + pa_ref[0] + pa_ref[1]
        hB = x + pb_ref[0] + pb_ref[1]
        fi = _mm(w1b[...], jnp.maximum(_mm(w1a[...], hA) + b1a_[...], 0.0)) \
            + b1b_[...]
        fo = _mm(w2b[...], jnp.maximum(_mm(w2a[...], hB) + b2a_[...], 0.0)) \
            + b2b_[...]
        cat = jnp.concatenate([x, fi, fo], axis=0)
        u = jnp.maximum(_mm(wf1[...], cat) + bf1_[...], 0.0)
        o_ref[...] = x + _mm(wf2[...], u) + bf2_[...]

    full = lambda shape: pl.BlockSpec(shape, lambda j: tuple(0 for _ in shape))
    return pl.pallas_call(
        body,
        grid=(GRID,),
        in_specs=[
            pl.BlockSpec((EMB, BN), lambda j: (0, j)),
            pl.BlockSpec((2, EMB, BN), lambda j: (0, 0, j)),
            pl.BlockSpec((2, EMB, BN), lambda j: (0, 0, j)),
            full((EMB, EMB)), full((EMB, 1)),
            full((EMB, EMB)), full((EMB, 1)),
            full((EMB, EMB)), full((EMB, 1)),
            full((EMB, EMB)), full((EMB, 1)),
            full((2 * EMB, 3 * EMB)), full((2 * EMB, 1)),
            full((EMB, 2 * EMB)), full((EMB, 1)),
        ],
        out_specs=pl.BlockSpec((EMB, BN), lambda j: (0, j)),
        out_shape=jax.ShapeDtypeStruct((EMB, NPAD), jnp.float32),
    )(x_t, pA, pB, W1a, b1a, W1b, b1b, W2a, b2a, W2b, b2b,
      Wf1, bf1, Wf2, bf2)


def _tc_final(x_t, Wc, bc_row, batch_row):
    DOUT = Wc.shape[0]

    def body(x_ref, wc_ref, bc_ref, bt_ref, o_ref, mx, sm, cn):
        j = pl.program_id(0)

        @pl.when(j == 0)
        def _():
            mx[...] = jnp.full((NUM_GRAPHS, DOUT), -jnp.inf, jnp.float32)
            sm[...] = jnp.zeros((NUM_GRAPHS, DOUT), jnp.float32)
            cn[...] = jnp.zeros((NUM_GRAPHS, 128), jnp.float32)

        yt = lax.dot_general(x_ref[...], wc_ref[...],
                             (((0,), (1,)), ((), ())),
                             preferred_element_type=jnp.float32) + bc_ref[...]
        bt = bt_ref[...]
        gids = lax.broadcasted_iota(jnp.int32, (NUM_GRAPHS, BN), 0)
        masks = (gids == bt).astype(jnp.float32)
        sm[...] += lax.dot_general(masks, yt, (((1,), (0,)), ((), ())),
                                   preferred_element_type=jnp.float32)
        cn[...] += jnp.sum(masks, axis=1, keepdims=True)
        for gph in range(NUM_GRAPHS):
            m = bt == gph
            ym = jnp.where(jnp.transpose(m), yt, -jnp.inf)
            gm = jnp.max(ym, axis=0, keepdims=True)
            mx[pl.ds(gph, 1), :] = jnp.maximum(mx[pl.ds(gph, 1), :], gm)

        @pl.when(j == GRID - 1)
        def _():
            o_ref[:, :DOUT] = mx[...]
            o_ref[:, DOUT:] = sm[...] / jnp.maximum(cn[:, :1], 1.0)

    return pl.pallas_call(
        body,
        grid=(GRID,),
        in_specs=[
            pl.BlockSpec((EMB, BN), lambda j: (0, j)),
            pl.BlockSpec((DOUT, EMB), lambda j: (0, 0)),
            pl.BlockSpec((1, DOUT), lambda j: (0, 0)),
            pl.BlockSpec((1, BN), lambda j: (0, j)),
        ],
        out_specs=pl.BlockSpec((NUM_GRAPHS, 2 * DOUT), lambda j: (0, 0)),
        out_shape=jax.ShapeDtypeStruct((NUM_GRAPHS, 2 * DOUT), jnp.float32),
        scratch_shapes=[
            pltpu.VMEM((NUM_GRAPHS, DOUT), jnp.float32),
            pltpu.VMEM((NUM_GRAPHS, DOUT), jnp.float32),
            pltpu.VMEM((NUM_GRAPHS, 128), jnp.float32),
        ],
    )(x_t, Wc, bc_row, batch_row)


def kernel(nodes, edges, edge_attr, batch,
           W_enc, b_enc, W_edge, b_edge,
           W1a, b1a, W1b, b1b, W2a, b2a, W2b, b2b,
           Wf1, bf1, Wf2, bf2, Wc, bc):
    src = edges[0]
    dst = edges[1]
    wv = W_edge[:, 0]
    bv = b_edge

    nodes_pad = jnp.pad(nodes, ((0, NPAD - N), (0, 0)))
    batch_row = jnp.pad(batch, (0, NPAD - N),
                        constant_values=NUM_GRAPHS).reshape(1, NPAD)

    x_t = _tc_encode(nodes_pad, W_enc, b_enc.reshape(EMB, 1))
    for _ in range(3):
        pA, pB = _sc_aggregate(x_t, src, dst, attr=edge_attr, wv=wv, bv=bv)
        x_t = _tc_iter(x_t, pA, pB,
                       W1a, b1a.reshape(EMB, 1), W1b, b1b.reshape(EMB, 1),
                       W2a, b2a.reshape(EMB, 1), W2b, b2b.reshape(EMB, 1),
                       Wf1, bf1.reshape(2 * EMB, 1),
                       Wf2, bf2.reshape(EMB, 1))
    return _tc_final(x_t, Wc, bc.reshape(1, 2 * EMB), batch_row)


# speedup vs baseline: 4.3869x; 1.7591x over previous
import dataclasses
import functools

import jax
import jax.numpy as jnp
from jax import lax
from jax.experimental import pallas as pl
from jax.experimental.pallas import tpu as pltpu
from jax.experimental.pallas import tpu_sc as plsc

N = 10000
NPAD = 10240
E = 320000
EMB = 64
NUM_GRAPHS = 16

C = 640
NPT = (E // 2) // C
FPT = 4
BN = 1024
GRID = NPAD // BN

_SC_MESH = plsc.VectorSubcoreMesh(core_axis_name="c", subcore_axis_name="s")

_SC_PARAMS = pltpu.CompilerParams()
if "needs_layout_passes" in pltpu.CompilerParams.__dataclass_fields__:
    _SC_PARAMS = dataclasses.replace(_SC_PARAMS, needs_layout_passes=False)


def _sc_aggregate(x_t, epack, wv, bv):
    out_t = [jax.ShapeDtypeStruct((2, EMB, NPAD), jnp.float32),
             jax.ShapeDtypeStruct((2, EMB, NPAD), jnp.float32)]

    @functools.partial(
        pl.kernel,
        out_type=out_t,
        mesh=_SC_MESH,
        compiler_params=_SC_PARAMS,
        scratch_types=[
            pltpu.VMEM((FPT, NPAD), jnp.float32),
            pltpu.VMEM((FPT, NPAD), jnp.float32),
            pltpu.VMEM((FPT, NPAD), jnp.float32),
            pltpu.VMEM((3, C), jnp.int32),
            pltpu.VMEM((3, C), jnp.int32),
            pltpu.VMEM((EMB,), jnp.float32),
            pltpu.VMEM((EMB,), jnp.float32),
            pltpu.SemaphoreType.DMA,
            pltpu.SemaphoreType.DMA,
        ],
    )
    def k(x_hbm, e_hbm, w_hbm, b_hbm, outA, outB,
          xv, aggA, aggB, ev0, ev1, wvv, bvv, sem0, sem1):
        cid = lax.axis_index("c")
        sid = lax.axis_index("s")
        wid = cid * 16 + sid
        g = wid % 16
        half = wid // 16
        ebase = half * (E // 2)

        pltpu.sync_copy(x_hbm.at[pl.ds(g * FPT, FPT)], xv)
        pltpu.sync_copy(w_hbm, wvv)
        pltpu.sync_copy(b_hbm, bvv)

        zero = jnp.zeros((16,), jnp.float32)

        @pl.loop(0, NPAD // 16)
        def _(i):
            for f in range(FPT):
                aggA[f, pl.ds(i * 16, 16)] = zero
                aggB[f, pl.ds(i * 16, 16)] = zero

        wb = [plsc.load_gather(wvv, [jnp.full((16,), g * FPT + f, jnp.int32)])
              for f in range(FPT)]
        bb = [plsc.load_gather(bvv, [jnp.full((16,), g * FPT + f, jnp.int32)])
              for f in range(FPT)]

        def issue(ci, ebuf, sem):
            pltpu.async_copy(e_hbm.at[:, pl.ds(ebase + ci * C, C)], ebuf, sem)

        def drain(ci, ebuf, sem):
            pltpu.make_async_copy(
                e_hbm.at[:, pl.ds(ebase + ci * C, C)], ebuf, sem).wait()

        def compute(ebuf):
            @pl.loop(0, C // 16)
            def _(i):
                sl = pl.ds(i * 16, 16)
                sv = ebuf[0, sl]
                dv = ebuf[1, sl]
                av = plsc.bitcast(ebuf[2, sl], jnp.float32)
                for f in range(FPT):
                    fidx = jnp.full((16,), f, jnp.int32)
                    ee = av * wb[f] + bb[f]
                    xa = plsc.load_gather(xv, [fidx, sv])
                    plsc.addupdate_scatter(aggA, [fidx, dv],
                                           jnp.maximum(xa + ee, 0.0))
                    xb = plsc.load_gather(xv, [fidx, dv])
                    plsc.addupdate_scatter(aggB, [fidx, sv],
                                           jnp.maximum(xb + ee, 0.0))

        issue(0, ev0, sem0)

        @pl.loop(0, NPT, step=2)
        def _(ci):
            drain(ci, ev0, sem0)
            issue(ci + 1, ev1, sem1)
            compute(ev0)
            drain(ci + 1, ev1, sem1)

            @pl.when(ci + 2 < NPT)
            def _():
                issue(ci + 2, ev0, sem0)

            compute(ev1)

        pltpu.sync_copy(aggA, outA.at[half, pl.ds(g * FPT, FPT)])
        pltpu.sync_copy(aggB, outB.at[half, pl.ds(g * FPT, FPT)])

    return k(x_t, epack, wv, bv)


def _mm(a, b):
    return lax.dot_general(a, b, (((1,), (0,)), ((), ())),
                           preferred_element_type=jnp.float32)


def _tc_encode(nodes_pad, W_enc, b_enc_col):
    d_in = nodes_pad.shape[1]

    def body(n_ref, w_ref, b_ref, o_ref):
        o_ref[...] = lax.dot_general(
            w_ref[...], n_ref[...], (((1,), (1,)), ((), ())),
            preferred_element_type=jnp.float32) + b_ref[...]

    return pl.pallas_call(
        body,
        grid=(GRID,),
        in_specs=[
            pl.BlockSpec((BN, d_in), lambda j: (j, 0)),
            pl.BlockSpec((EMB, d_in), lambda j: (0, 0)),
            pl.BlockSpec((EMB, 1), lambda j: (0, 0)),
        ],
        out_specs=pl.BlockSpec((EMB, BN), lambda j: (0, j)),
        out_shape=jax.ShapeDtypeStruct((EMB, NPAD), jnp.float32),
    )(nodes_pad, W_enc, b_enc_col)


def _tc_iter(x_t, pA, pB, W1a, b1a, W1b, b1b, W2a, b2a, W2b, b2b,
             Wf1, bf1, Wf2, bf2):
    def body(x_ref, pa_ref, pb_ref,
             w1a, b1a_, w1b, b1b_, w2a, b2a_, w2b, b2b_,
             wf1, bf1_, wf2, bf2_, o_ref):
        x = x_ref[...]
        hA = x + pa_ref[0] + pa_ref[1]
        hB = x + pb_ref[0] + pb_ref[1]
        fi = _mm(w1b[...], jnp.maximum(_mm(w1a[...], hA) + b1a_[...], 0.0)) \
            + b1b_[...]
        fo = _mm(w2b[...], jnp.maximum(_mm(w2a[...], hB) + b2a_[...], 0.0)) \
            + b2b_[...]
        cat = jnp.concatenate([x, fi, fo], axis=0)
        u = jnp.maximum(_mm(wf1[...], cat) + bf1_[...], 0.0)
        o_ref[...] = x + _mm(wf2[...], u) + bf2_[...]

    full = lambda shape: pl.BlockSpec(shape, lambda j: tuple(0 for _ in shape))
    return pl.pallas_call(
        body,
        grid=(GRID,),
        in_specs=[
            pl.BlockSpec((EMB, BN), lambda j: (0, j)),
            pl.BlockSpec((2, EMB, BN), lambda j: (0, 0, j)),
            pl.BlockSpec((2, EMB, BN), lambda j: (0, 0, j)),
            full((EMB, EMB)), full((EMB, 1)),
            full((EMB, EMB)), full((EMB, 1)),
            full((EMB, EMB)), full((EMB, 1)),
            full((EMB, EMB)), full((EMB, 1)),
            full((2 * EMB, 3 * EMB)), full((2 * EMB, 1)),
            full((EMB, 2 * EMB)), full((EMB, 1)),
        ],
        out_specs=pl.BlockSpec((EMB, BN), lambda j: (0, j)),
        out_shape=jax.ShapeDtypeStruct((EMB, NPAD), jnp.float32),
    )(x_t, pA, pB, W1a, b1a, W1b, b1b, W2a, b2a, W2b, b2b,
      Wf1, bf1, Wf2, bf2)


def _tc_final(x_t, Wc, bc_row, batch_row):
    DOUT = Wc.shape[0]

    def body(x_ref, wc_ref, bc_ref, bt_ref, o_ref, mx, sm, cn):
        j = pl.program_id(0)

        @pl.when(j == 0)
        def _():
            mx[...] = jnp.full((NUM_GRAPHS, DOUT), -jnp.inf, jnp.float32)
            sm[...] = jnp.zeros((NUM_GRAPHS, DOUT), jnp.float32)
            cn[...] = jnp.zeros((NUM_GRAPHS, 128), jnp.float32)

        yt = lax.dot_general(x_ref[...], wc_ref[...],
                             (((0,), (1,)), ((), ())),
                             preferred_element_type=jnp.float32) + bc_ref[...]
        bt = bt_ref[...]
        gids = lax.broadcasted_iota(jnp.int32, (NUM_GRAPHS, BN), 0)
        masks = (gids == bt).astype(jnp.float32)
        sm[...] += lax.dot_general(masks, yt, (((1,), (0,)), ((), ())),
                                   preferred_element_type=jnp.float32)
        cn[...] += jnp.sum(masks, axis=1, keepdims=True)
        for gph in range(NUM_GRAPHS):
            m = bt == gph
            ym = jnp.where(jnp.transpose(m), yt, -jnp.inf)
            gm = jnp.max(ym, axis=0, keepdims=True)
            mx[pl.ds(gph, 1), :] = jnp.maximum(mx[pl.ds(gph, 1), :], gm)

        @pl.when(j == GRID - 1)
        def _():
            o_ref[:, :DOUT] = mx[...]
            o_ref[:, DOUT:] = sm[...] / jnp.maximum(cn[:, :1], 1.0)

    return pl.pallas_call(
        body,
        grid=(GRID,),
        in_specs=[
            pl.BlockSpec((EMB, BN), lambda j: (0, j)),
            pl.BlockSpec((DOUT, EMB), lambda j: (0, 0)),
            pl.BlockSpec((1, DOUT), lambda j: (0, 0)),
            pl.BlockSpec((1, BN), lambda j: (0, j)),
        ],
        out_specs=pl.BlockSpec((NUM_GRAPHS, 2 * DOUT), lambda j: (0, 0)),
        out_shape=jax.ShapeDtypeStruct((NUM_GRAPHS, 2 * DOUT), jnp.float32),
        scratch_shapes=[
            pltpu.VMEM((NUM_GRAPHS, DOUT), jnp.float32),
            pltpu.VMEM((NUM_GRAPHS, DOUT), jnp.float32),
            pltpu.VMEM((NUM_GRAPHS, 128), jnp.float32),
        ],
    )(x_t, Wc, bc_row, batch_row)


def kernel(nodes, edges, edge_attr, batch,
           W_enc, b_enc, W_edge, b_edge,
           W1a, b1a, W1b, b1b, W2a, b2a, W2b, b2b,
           Wf1, bf1, Wf2, bf2, Wc, bc):
    attr_bits = lax.bitcast_convert_type(edge_attr, jnp.int32)
    epack = jnp.concatenate(
        [edges[0][None], edges[1][None], attr_bits[None]], axis=0)
    wv = W_edge[:, 0]
    bv = b_edge

    nodes_pad = jnp.pad(nodes, ((0, NPAD - N), (0, 0)))
    batch_row = jnp.pad(batch, (0, NPAD - N),
                        constant_values=NUM_GRAPHS).reshape(1, NPAD)

    x_t = _tc_encode(nodes_pad, W_enc, b_enc.reshape(EMB, 1))
    for _ in range(3):
        pA, pB = _sc_aggregate(x_t, epack, wv, bv)
        x_t = _tc_iter(x_t, pA, pB,
                       W1a, b1a.reshape(EMB, 1), W1b, b1b.reshape(EMB, 1),
                       W2a, b2a.reshape(EMB, 1), W2b, b2b.reshape(EMB, 1),
                       Wf1, bf1.reshape(2 * EMB, 1),
                       Wf2, bf2.reshape(EMB, 1))
    return _tc_final(x_t, Wc, bc.reshape(1, 2 * EMB), batch_row)


# speedup vs baseline: 8.8619x; 2.0201x over previous
import dataclasses
import functools

import jax
import jax.numpy as jnp
from jax import lax
from jax.experimental import pallas as pl
from jax.experimental.pallas import tpu as pltpu
from jax.experimental.pallas import tpu_sc as plsc

N = 10000
NPAD = 10240
E = 320000
EMB = 64
NUM_GRAPHS = 16

C = 640
NPT = (E // 2) // C
FPT = 4
BN = 1024
GRID = NPAD // BN

_SC_MESH = plsc.VectorSubcoreMesh(core_axis_name="c", subcore_axis_name="s")

_SC_PARAMS = pltpu.CompilerParams()
if "needs_layout_passes" in pltpu.CompilerParams.__dataclass_fields__:
    _SC_PARAMS = dataclasses.replace(_SC_PARAMS, needs_layout_passes=False)


def _sc_aggregate(x_t, epack, wv, bv):
    out_t = [jax.ShapeDtypeStruct((2, EMB, NPAD), jnp.float32),
             jax.ShapeDtypeStruct((2, EMB, NPAD), jnp.float32)]

    @functools.partial(
        pl.kernel,
        out_type=out_t,
        mesh=_SC_MESH,
        compiler_params=_SC_PARAMS,
        scratch_types=[
            pltpu.VMEM((FPT, NPAD), jnp.float32),
            pltpu.VMEM((FPT, NPAD), jnp.float32),
            pltpu.VMEM((FPT, NPAD), jnp.float32),
            pltpu.VMEM((3, C), jnp.int32),
            pltpu.VMEM((3, C), jnp.int32),
            pltpu.VMEM((EMB,), jnp.float32),
            pltpu.VMEM((EMB,), jnp.float32),
            pltpu.SemaphoreType.DMA,
            pltpu.SemaphoreType.DMA,
        ],
    )
    def k(x_hbm, e_hbm, w_hbm, b_hbm, outA, outB,
          xv, aggA, aggB, ev0, ev1, wvv, bvv, sem0, sem1):
        cid = lax.axis_index("c")
        sid = lax.axis_index("s")
        wid = cid * 16 + sid
        g = wid % 16
        half = wid // 16
        ebase = half * (E // 2)

        pltpu.sync_copy(x_hbm.at[pl.ds(g * FPT, FPT)], xv)
        pltpu.sync_copy(w_hbm, wvv)
        pltpu.sync_copy(b_hbm, bvv)

        zero = jnp.zeros((16,), jnp.float32)

        @pl.loop(0, NPAD // 16)
        def _(i):
            for f in range(FPT):
                aggA[f, pl.ds(i * 16, 16)] = zero
                aggB[f, pl.ds(i * 16, 16)] = zero

        wb = [plsc.load_gather(wvv, [jnp.full((16,), g * FPT + f, jnp.int32)])
              for f in range(FPT)]
        bb = [plsc.load_gather(bvv, [jnp.full((16,), g * FPT + f, jnp.int32)])
              for f in range(FPT)]

        def issue(ci, ebuf, sem):
            pltpu.async_copy(e_hbm.at[:, pl.ds(ebase + ci * C, C)], ebuf, sem)

        def drain(ci, ebuf, sem):
            pltpu.make_async_copy(
                e_hbm.at[:, pl.ds(ebase + ci * C, C)], ebuf, sem).wait()

        def compute(ebuf):
            @pl.loop(0, C // 16)
            def _(i):
                sl = pl.ds(i * 16, 16)
                sv = ebuf[0, sl]
                dv = ebuf[1, sl]
                av = plsc.bitcast(ebuf[2, sl], jnp.float32)
                fidx = [jnp.full((16,), f, jnp.int32) for f in range(FPT)]
                xa = [plsc.load_gather(xv, [fidx[f], sv]) for f in range(FPT)]
                xb = [plsc.load_gather(xv, [fidx[f], dv]) for f in range(FPT)]
                ee = [av * wb[f] + bb[f] for f in range(FPT)]
                for f in range(FPT):
                    plsc.addupdate_scatter(aggA, [fidx[f], dv],
                                           jnp.maximum(xa[f] + ee[f], 0.0))
                for f in range(FPT):
                    plsc.addupdate_scatter(aggB, [fidx[f], sv],
                                           jnp.maximum(xb[f] + ee[f], 0.0))

        issue(0, ev0, sem0)

        @pl.loop(0, NPT, step=2)
        def _(ci):
            drain(ci, ev0, sem0)
            issue(ci + 1, ev1, sem1)
            compute(ev0)
            drain(ci + 1, ev1, sem1)

            @pl.when(ci + 2 < NPT)
            def _():
                issue(ci + 2, ev0, sem0)

            compute(ev1)

        pltpu.sync_copy(aggA, outA.at[half, pl.ds(g * FPT, FPT)])
        pltpu.sync_copy(aggB, outB.at[half, pl.ds(g * FPT, FPT)])

    return k(x_t, epack, wv, bv)


def _mm(a, b):
    return lax.dot_general(a, b, (((1,), (0,)), ((), ())),
                           preferred_element_type=jnp.float32)


def _tc_encode(nodes_pad, W_enc, b_enc_col):
    d_in = nodes_pad.shape[1]

    def body(n_ref, w_ref, b_ref, o_ref):
        o_ref[...] = lax.dot_general(
            w_ref[...], n_ref[...], (((1,), (1,)), ((), ())),
            preferred_element_type=jnp.float32) + b_ref[...]

    return pl.pallas_call(
        body,
        grid=(GRID,),
        in_specs=[
            pl.BlockSpec((BN, d_in), lambda j: (j, 0)),
            pl.BlockSpec((EMB, d_in), lambda j: (0, 0)),
            pl.BlockSpec((EMB, 1), lambda j: (0, 0)),
        ],
        out_specs=pl.BlockSpec((EMB, BN), lambda j: (0, j)),
        out_shape=jax.ShapeDtypeStruct((EMB, NPAD), jnp.float32),
    )(nodes_pad, W_enc, b_enc_col)


def _tc_iter(x_t, pA, pB, W1a, b1a, W1b, b1b, W2a, b2a, W2b, b2b,
             Wf1, bf1, Wf2, bf2):
    def body(x_ref, pa_ref, pb_ref,
             w1a, b1a_, w1b, b1b_, w2a, b2a_, w2b, b2b_,
             wf1, bf1_, wf2, bf2_, o_ref):
        x = x_ref[...]
        hA = x + pa_ref[0] + pa_ref[1]
        hB = x + pb_ref[0] + pb_ref[1]
        fi = _mm(w1b[...], jnp.maximum(_mm(w1a[...], hA) + b1a_[...], 0.0)) \
            + b1b_[...]
        fo = _mm(w2b[...], jnp.maximum(_mm(w2a[...], hB) + b2a_[...], 0.0)) \
            + b2b_[...]
        cat = jnp.concatenate([x, fi, fo], axis=0)
        u = jnp.maximum(_mm(wf1[...], cat) + bf1_[...], 0.0)
        o_ref[...] = x + _mm(wf2[...], u) + bf2_[...]

    full = lambda shape: pl.BlockSpec(shape, lambda j: tuple(0 for _ in shape))
    return pl.pallas_call(
        body,
        grid=(GRID,),
        in_specs=[
            pl.BlockSpec((EMB, BN), lambda j: (0, j)),
            pl.BlockSpec((2, EMB, BN), lambda j: (0, 0, j)),
            pl.BlockSpec((2, EMB, BN), lambda j: (0, 0, j)),
            full((EMB, EMB)), full((EMB, 1)),
            full((EMB, EMB)), full((EMB, 1)),
            full((EMB, EMB)), full((EMB, 1)),
            full((EMB, EMB)), full((EMB, 1)),
            full((2 * EMB, 3 * EMB)), full((2 * EMB, 1)),
            full((EMB, 2 * EMB)), full((EMB, 1)),
        ],
        out_specs=pl.BlockSpec((EMB, BN), lambda j: (0, j)),
        out_shape=jax.ShapeDtypeStruct((EMB, NPAD), jnp.float32),
    )(x_t, pA, pB, W1a, b1a, W1b, b1b, W2a, b2a, W2b, b2b,
      Wf1, bf1, Wf2, bf2)


def _tc_final(x_t, Wc, bc_row, batch_row):
    DOUT = Wc.shape[0]

    def body(x_ref, wc_ref, bc_ref, bt_ref, o_ref, mx, sm, cn):
        j = pl.program_id(0)

        @pl.when(j == 0)
        def _():
            mx[...] = jnp.full((NUM_GRAPHS, DOUT), -jnp.inf, jnp.float32)
            sm[...] = jnp.zeros((NUM_GRAPHS, DOUT), jnp.float32)
            cn[...] = jnp.zeros((NUM_GRAPHS, 128), jnp.float32)

        yt = lax.dot_general(x_ref[...], wc_ref[...],
                             (((0,), (1,)), ((), ())),
                             preferred_element_type=jnp.float32) + bc_ref[...]
        bt = bt_ref[...]
        gids = lax.broadcasted_iota(jnp.int32, (NUM_GRAPHS, BN), 0)
        masks = (gids == bt).astype(jnp.float32)
        sm[...] += lax.dot_general(masks, yt, (((1,), (0,)), ((), ())),
                                   preferred_element_type=jnp.float32)
        cn[...] += jnp.sum(masks, axis=1, keepdims=True)
        for gph in range(NUM_GRAPHS):
            m = bt == gph
            ym = jnp.where(jnp.transpose(m), yt, -jnp.inf)
            gm = jnp.max(ym, axis=0, keepdims=True)
            mx[pl.ds(gph, 1), :] = jnp.maximum(mx[pl.ds(gph, 1), :], gm)

        @pl.when(j == GRID - 1)
        def _():
            o_ref[:, :DOUT] = mx[...]
            o_ref[:, DOUT:] = sm[...] / jnp.maximum(cn[:, :1], 1.0)

    return pl.pallas_call(
        body,
        grid=(GRID,),
        in_specs=[
            pl.BlockSpec((EMB, BN), lambda j: (0, j)),
            pl.BlockSpec((DOUT, EMB), lambda j: (0, 0)),
            pl.BlockSpec((1, DOUT), lambda j: (0, 0)),
            pl.BlockSpec((1, BN), lambda j: (0, j)),
        ],
        out_specs=pl.BlockSpec((NUM_GRAPHS, 2 * DOUT), lambda j: (0, 0)),
        out_shape=jax.ShapeDtypeStruct((NUM_GRAPHS, 2 * DOUT), jnp.float32),
        scratch_shapes=[
            pltpu.VMEM((NUM_GRAPHS, DOUT), jnp.float32),
            pltpu.VMEM((NUM_GRAPHS, DOUT), jnp.float32),
            pltpu.VMEM((NUM_GRAPHS, 128), jnp.float32),
        ],
    )(x_t, Wc, bc_row, batch_row)


def kernel(nodes, edges, edge_attr, batch,
           W_enc, b_enc, W_edge, b_edge,
           W1a, b1a, W1b, b1b, W2a, b2a, W2b, b2b,
           Wf1, bf1, Wf2, bf2, Wc, bc):
    attr_bits = lax.bitcast_convert_type(edge_attr, jnp.int32)
    epack = jnp.concatenate(
        [edges[0][None], edges[1][None], attr_bits[None]], axis=0)
    wv = W_edge[:, 0]
    bv = b_edge

    nodes_pad = jnp.pad(nodes, ((0, NPAD - N), (0, 0)))
    batch_row = jnp.pad(batch, (0, NPAD - N),
                        constant_values=NUM_GRAPHS).reshape(1, NPAD)

    x_t = _tc_encode(nodes_pad, W_enc, b_enc.reshape(EMB, 1))
    for _ in range(3):
        pA, pB = _sc_aggregate(x_t, epack, wv, bv)
        x_t = _tc_iter(x_t, pA, pB,
                       W1a, b1a.reshape(EMB, 1), W1b, b1b.reshape(EMB, 1),
                       W2a, b2a.reshape(EMB, 1), W2b, b2b.reshape(EMB, 1),
                       Wf1, bf1.reshape(2 * EMB, 1),
                       Wf2, bf2.reshape(EMB, 1))
    return _tc_final(x_t, Wc, bc.reshape(1, 2 * EMB), batch_row)


# speedup vs baseline: 8.9529x; 1.0103x over previous
import dataclasses
import functools

import jax
import jax.numpy as jnp
from jax import lax
from jax.experimental import pallas as pl
from jax.experimental.pallas import tpu as pltpu
from jax.experimental.pallas import tpu_sc as plsc

N = 10000
NPAD = 10240
E = 320000
EMB = 64
NUM_GRAPHS = 16

C = 640
NPT = (E // 2) // C
FPT = 4
BN = 1024
GRID = NPAD // BN

_SC_MESH = plsc.VectorSubcoreMesh(core_axis_name="c", subcore_axis_name="s")

_SC_PARAMS = pltpu.CompilerParams()
if "needs_layout_passes" in pltpu.CompilerParams.__dataclass_fields__:
    _SC_PARAMS = dataclasses.replace(_SC_PARAMS, needs_layout_passes=False)


def _sc_aggregate(x_t, epack, wv, bv):
    out_t = [jax.ShapeDtypeStruct((2, EMB, NPAD), jnp.float32),
             jax.ShapeDtypeStruct((2, EMB, NPAD), jnp.float32)]

    @functools.partial(
        pl.kernel,
        out_type=out_t,
        mesh=_SC_MESH,
        compiler_params=_SC_PARAMS,
        scratch_types=[
            pltpu.VMEM((FPT, NPAD), jnp.float32),
            pltpu.VMEM((FPT, NPAD), jnp.float32),
            pltpu.VMEM((FPT, NPAD), jnp.float32),
            pltpu.VMEM((3, C), jnp.int32),
            pltpu.VMEM((3, C), jnp.int32),
            pltpu.VMEM((EMB,), jnp.float32),
            pltpu.VMEM((EMB,), jnp.float32),
            pltpu.SemaphoreType.DMA,
            pltpu.SemaphoreType.DMA,
        ],
    )
    def k(x_hbm, e_hbm, w_hbm, b_hbm, outA, outB,
          xv, aggA, aggB, ev0, ev1, wvv, bvv, sem0, sem1):
        cid = lax.axis_index("c")
        sid = lax.axis_index("s")
        wid = cid * 16 + sid
        g = wid % 16
        half = wid // 16
        ebase = half * (E // 2)

        pltpu.sync_copy(x_hbm.at[pl.ds(g * FPT, FPT)], xv)
        pltpu.sync_copy(w_hbm, wvv)
        pltpu.sync_copy(b_hbm, bvv)

        zero = jnp.zeros((16,), jnp.float32)

        @pl.loop(0, NPAD // 16)
        def _(i):
            for f in range(FPT):
                aggA[f, pl.ds(i * 16, 16)] = zero
                aggB[f, pl.ds(i * 16, 16)] = zero

        wb = [plsc.load_gather(wvv, [jnp.full((16,), g * FPT + f, jnp.int32)])
              for f in range(FPT)]
        bb = [plsc.load_gather(bvv, [jnp.full((16,), g * FPT + f, jnp.int32)])
              for f in range(FPT)]

        def issue(ci, ebuf, sem):
            pltpu.async_copy(e_hbm.at[:, pl.ds(ebase + ci * C, C)], ebuf, sem)

        def drain(ci, ebuf, sem):
            pltpu.make_async_copy(
                e_hbm.at[:, pl.ds(ebase + ci * C, C)], ebuf, sem).wait()

        def compute(ebuf):
            def step(sl):
                sv = ebuf[0, sl]
                dv = ebuf[1, sl]
                av = plsc.bitcast(ebuf[2, sl], jnp.float32)
                fidx = [jnp.full((16,), f, jnp.int32) for f in range(FPT)]
                xa = [plsc.load_gather(xv, [fidx[f], sv]) for f in range(FPT)]
                xb = [plsc.load_gather(xv, [fidx[f], dv]) for f in range(FPT)]
                ee = [av * wb[f] + bb[f] for f in range(FPT)]
                for f in range(FPT):
                    plsc.addupdate_scatter(aggA, [fidx[f], dv],
                                           jnp.maximum(xa[f] + ee[f], 0.0))
                for f in range(FPT):
                    plsc.addupdate_scatter(aggB, [fidx[f], sv],
                                           jnp.maximum(xb[f] + ee[f], 0.0))

            @pl.loop(0, C // 16, step=2)
            def _(i):
                step(pl.ds(i * 16, 16))
                step(pl.ds(i * 16 + 16, 16))

        issue(0, ev0, sem0)

        @pl.loop(0, NPT, step=2)
        def _(ci):
            drain(ci, ev0, sem0)
            issue(ci + 1, ev1, sem1)
            compute(ev0)
            drain(ci + 1, ev1, sem1)

            @pl.when(ci + 2 < NPT)
            def _():
                issue(ci + 2, ev0, sem0)

            compute(ev1)

        pltpu.sync_copy(aggA, outA.at[half, pl.ds(g * FPT, FPT)])
        pltpu.sync_copy(aggB, outB.at[half, pl.ds(g * FPT, FPT)])

    return k(x_t, epack, wv, bv)


def _mm(a, b):
    return lax.dot_general(a, b, (((1,), (0,)), ((), ())),
                           preferred_element_type=jnp.float32)


def _tc_encode(nodes_pad, W_enc, b_enc_col):
    d_in = nodes_pad.shape[1]

    def body(n_ref, w_ref, b_ref, o_ref):
        o_ref[...] = lax.dot_general(
            w_ref[...], n_ref[...], (((1,), (1,)), ((), ())),
            preferred_element_type=jnp.float32) + b_ref[...]

    return pl.pallas_call(
        body,
        grid=(GRID,),
        in_specs=[
            pl.BlockSpec((BN, d_in), lambda j: (j, 0)),
            pl.BlockSpec((EMB, d_in), lambda j: (0, 0)),
            pl.BlockSpec((EMB, 1), lambda j: (0, 0)),
        ],
        out_specs=pl.BlockSpec((EMB, BN), lambda j: (0, j)),
        out_shape=jax.ShapeDtypeStruct((EMB, NPAD), jnp.float32),
    )(nodes_pad, W_enc, b_enc_col)


def _tc_iter(x_t, pA, pB, W1a, b1a, W1b, b1b, W2a, b2a, W2b, b2b,
             Wf1, bf1, Wf2, bf2):
    def body(x_ref, pa_ref, pb_ref,
             w1a, b1a_, w1b, b1b_, w2a, b2a_, w2b, b2b_,
             wf1, bf1_, wf2, bf2_, o_ref):
        x = x_ref[...]
        hA = x + pa_ref[0] + pa_ref[1]
        hB = x + pb_ref[0] + pb_ref[1]
        fi = _mm(w1b[...], jnp.maximum(_mm(w1a[...], hA) + b1a_[...], 0.0)) \
            + b1b_[...]
        fo = _mm(w2b[...], jnp.maximum(_mm(w2a[...], hB) + b2a_[...], 0.0)) \
            + b2b_[...]
        cat = jnp.concatenate([x, fi, fo], axis=0)
        u = jnp.maximum(_mm(wf1[...], cat) + bf1_[...], 0.0)
        o_ref[...] = x + _mm(wf2[...], u) + bf2_[...]

    full = lambda shape: pl.BlockSpec(shape, lambda j: tuple(0 for _ in shape))
    return pl.pallas_call(
        body,
        grid=(GRID,),
        in_specs=[
            pl.BlockSpec((EMB, BN), lambda j: (0, j)),
            pl.BlockSpec((2, EMB, BN), lambda j: (0, 0, j)),
            pl.BlockSpec((2, EMB, BN), lambda j: (0, 0, j)),
            full((EMB, EMB)), full((EMB, 1)),
            full((EMB, EMB)), full((EMB, 1)),
            full((EMB, EMB)), full((EMB, 1)),
            full((EMB, EMB)), full((EMB, 1)),
            full((2 * EMB, 3 * EMB)), full((2 * EMB, 1)),
            full((EMB, 2 * EMB)), full((EMB, 1)),
        ],
        out_specs=pl.BlockSpec((EMB, BN), lambda j: (0, j)),
        out_shape=jax.ShapeDtypeStruct((EMB, NPAD), jnp.float32),
    )(x_t, pA, pB, W1a, b1a, W1b, b1b, W2a, b2a, W2b, b2b,
      Wf1, bf1, Wf2, bf2)


def _tc_final(x_t, Wc, bc_row, batch_row):
    DOUT = Wc.shape[0]

    def body(x_ref, wc_ref, bc_ref, bt_ref, o_ref, mx, sm, cn):
        j = pl.program_id(0)

        @pl.when(j == 0)
        def _():
            mx[...] = jnp.full((NUM_GRAPHS, DOUT), -jnp.inf, jnp.float32)
            sm[...] = jnp.zeros((NUM_GRAPHS, DOUT), jnp.float32)
            cn[...] = jnp.zeros((NUM_GRAPHS, 128), jnp.float32)

        yt = lax.dot_general(x_ref[...], wc_ref[...],
                             (((0,), (1,)), ((), ())),
                             preferred_element_type=jnp.float32) + bc_ref[...]
        bt = bt_ref[...]
        gids = lax.broadcasted_iota(jnp.int32, (NUM_GRAPHS, BN), 0)
        masks = (gids == bt).astype(jnp.float32)
        sm[...] += lax.dot_general(masks, yt, (((1,), (0,)), ((), ())),
                                   preferred_element_type=jnp.float32)
        cn[...] += jnp.sum(masks, axis=1, keepdims=True)
        for gph in range(NUM_GRAPHS):
            m = bt == gph
            ym = jnp.where(jnp.transpose(m), yt, -jnp.inf)
            gm = jnp.max(ym, axis=0, keepdims=True)
            mx[pl.ds(gph, 1), :] = jnp.maximum(mx[pl.ds(gph, 1), :], gm)

        @pl.when(j == GRID - 1)
        def _():
            o_ref[:, :DOUT] = mx[...]
            o_ref[:, DOUT:] = sm[...] / jnp.maximum(cn[:, :1], 1.0)

    return pl.pallas_call(
        body,
        grid=(GRID,),
        in_specs=[
            pl.BlockSpec((EMB, BN), lambda j: (0, j)),
            pl.BlockSpec((DOUT, EMB), lambda j: (0, 0)),
            pl.BlockSpec((1, DOUT), lambda j: (0, 0)),
            pl.BlockSpec((1, BN), lambda j: (0, j)),
        ],
        out_specs=pl.BlockSpec((NUM_GRAPHS, 2 * DOUT), lambda j: (0, 0)),
        out_shape=jax.ShapeDtypeStruct((NUM_GRAPHS, 2 * DOUT), jnp.float32),
        scratch_shapes=[
            pltpu.VMEM((NUM_GRAPHS, DOUT), jnp.float32),
            pltpu.VMEM((NUM_GRAPHS, DOUT), jnp.float32),
            pltpu.VMEM((NUM_GRAPHS, 128), jnp.float32),
        ],
    )(x_t, Wc, bc_row, batch_row)


def kernel(nodes, edges, edge_attr, batch,
           W_enc, b_enc, W_edge, b_edge,
           W1a, b1a, W1b, b1b, W2a, b2a, W2b, b2b,
           Wf1, bf1, Wf2, bf2, Wc, bc):
    attr_bits = lax.bitcast_convert_type(edge_attr, jnp.int32)
    epack = jnp.concatenate(
        [edges[0][None], edges[1][None], attr_bits[None]], axis=0)
    wv = W_edge[:, 0]
    bv = b_edge

    nodes_pad = jnp.pad(nodes, ((0, NPAD - N), (0, 0)))
    batch_row = jnp.pad(batch, (0, NPAD - N),
                        constant_values=NUM_GRAPHS).reshape(1, NPAD)

    x_t = _tc_encode(nodes_pad, W_enc, b_enc.reshape(EMB, 1))
    for _ in range(3):
        pA, pB = _sc_aggregate(x_t, epack, wv, bv)
        x_t = _tc_iter(x_t, pA, pB,
                       W1a, b1a.reshape(EMB, 1), W1b, b1b.reshape(EMB, 1),
                       W2a, b2a.reshape(EMB, 1), W2b, b2b.reshape(EMB, 1),
                       Wf1, bf1.reshape(2 * EMB, 1),
                       Wf2, bf2.reshape(EMB, 1))
    return _tc_final(x_t, Wc, bc.reshape(1, 2 * EMB), batch_row)


# speedup vs baseline: 9.6684x; 1.0799x over previous
import dataclasses
import functools

import jax
import jax.numpy as jnp
from jax import lax
from jax.experimental import pallas as pl
from jax.experimental.pallas import tpu as pltpu
from jax.experimental.pallas import tpu_sc as plsc

N = 10000
NPAD = 10240
E = 320000
EMB = 64
NUM_GRAPHS = 16

C = 640
NPT = (E // 2) // C
FPT = 4
BN = 1024
GRID = NPAD // BN

_SC_MESH = plsc.VectorSubcoreMesh(core_axis_name="c", subcore_axis_name="s")

_SC_PARAMS = pltpu.CompilerParams()
if "needs_layout_passes" in pltpu.CompilerParams.__dataclass_fields__:
    _SC_PARAMS = dataclasses.replace(_SC_PARAMS, needs_layout_passes=False)


def _sc_aggregate(xb_t, epack, wv):
    out_t = [jax.ShapeDtypeStruct((2, EMB, NPAD), jnp.float32),
             jax.ShapeDtypeStruct((2, EMB, NPAD), jnp.float32)]

    row = pltpu.VMEM((NPAD,), jnp.float32)

    @functools.partial(
        pl.kernel,
        out_type=out_t,
        mesh=_SC_MESH,
        compiler_params=_SC_PARAMS,
        scratch_types=[
            [row] * FPT,
            [row] * FPT,
            [row] * FPT,
            pltpu.VMEM((3, C), jnp.int32),
            pltpu.VMEM((3, C), jnp.int32),
            pltpu.VMEM((EMB,), jnp.float32),
            pltpu.SemaphoreType.DMA,
            pltpu.SemaphoreType.DMA,
        ],
    )
    def k(x_hbm, e_hbm, w_hbm, outA, outB,
          xv, aggA, aggB, ev0, ev1, wvv, sem0, sem1):
        cid = lax.axis_index("c")
        sid = lax.axis_index("s")
        wid = cid * 16 + sid
        g = wid % 16
        half = wid // 16
        ebase = half * (E // 2)

        for f in range(FPT):
            pltpu.sync_copy(x_hbm.at[g * FPT + f], xv[f])
        pltpu.sync_copy(w_hbm, wvv)

        zero = jnp.zeros((16,), jnp.float32)

        @pl.loop(0, NPAD // 16)
        def _(i):
            sl = pl.ds(i * 16, 16)
            for f in range(FPT):
                aggA[f][sl] = zero
                aggB[f][sl] = zero

        wb = [plsc.load_gather(wvv, [jnp.full((16,), g * FPT + f, jnp.int32)])
              for f in range(FPT)]

        def issue(ci, ebuf, sem):
            pltpu.async_copy(e_hbm.at[:, pl.ds(ebase + ci * C, C)], ebuf, sem)

        def drain(ci, ebuf, sem):
            pltpu.make_async_copy(
                e_hbm.at[:, pl.ds(ebase + ci * C, C)], ebuf, sem).wait()

        def compute(ebuf):
            def step(sl):
                sv = ebuf[0, sl]
                dv = ebuf[1, sl]
                av = plsc.bitcast(ebuf[2, sl], jnp.float32)
                xa = [plsc.load_gather(xv[f], [sv]) for f in range(FPT)]
                xb = [plsc.load_gather(xv[f], [dv]) for f in range(FPT)]
                ee = [av * wb[f] for f in range(FPT)]
                for f in range(FPT):
                    plsc.addupdate_scatter(aggA[f], [dv],
                                           jnp.maximum(xa[f] + ee[f], 0.0))
                for f in range(FPT):
                    plsc.addupdate_scatter(aggB[f], [sv],
                                           jnp.maximum(xb[f] + ee[f], 0.0))

            @pl.loop(0, C // 16, step=2)
            def _(i):
                step(pl.ds(i * 16, 16))
                step(pl.ds(i * 16 + 16, 16))

        issue(0, ev0, sem0)

        @pl.loop(0, NPT, step=2)
        def _(ci):
            drain(ci, ev0, sem0)
            issue(ci + 1, ev1, sem1)
            compute(ev0)
            drain(ci + 1, ev1, sem1)

            @pl.when(ci + 2 < NPT)
            def _():
                issue(ci + 2, ev0, sem0)

            compute(ev1)

        for f in range(FPT):
            pltpu.sync_copy(aggA[f], outA.at[half, g * FPT + f])
            pltpu.sync_copy(aggB[f], outB.at[half, g * FPT + f])

    return k(xb_t, epack, wv)


def _mm(a, b):
    return lax.dot_general(a, b, (((1,), (0,)), ((), ())),
                           preferred_element_type=jnp.float32)


def _tc_encode(nodes_pad, W_enc, b_enc_col, bv_col):
    d_in = nodes_pad.shape[1]

    def body(n_ref, w_ref, b_ref, bv_ref, o_ref, ob_ref):
        o = lax.dot_general(
            w_ref[...], n_ref[...], (((1,), (1,)), ((), ())),
            preferred_element_type=jnp.float32) + b_ref[...]
        o_ref[...] = o
        ob_ref[...] = o + bv_ref[...]

    return pl.pallas_call(
        body,
        grid=(GRID,),
        in_specs=[
            pl.BlockSpec((BN, d_in), lambda j: (j, 0)),
            pl.BlockSpec((EMB, d_in), lambda j: (0, 0)),
            pl.BlockSpec((EMB, 1), lambda j: (0, 0)),
            pl.BlockSpec((EMB, 1), lambda j: (0, 0)),
        ],
        out_specs=[pl.BlockSpec((EMB, BN), lambda j: (0, j))] * 2,
        out_shape=[jax.ShapeDtypeStruct((EMB, NPAD), jnp.float32)] * 2,
    )(nodes_pad, W_enc, b_enc_col, bv_col)


def _tc_iter(x_t, pA, pB, W1a, b1a, W1b, b1b, W2a, b2a, W2b, b2b,
             Wf1, bf1, Wf2, bf2, bv_col):
    def body(x_ref, pa_ref, pb_ref,
             w1a, b1a_, w1b, b1b_, w2a, b2a_, w2b, b2b_,
             wf1, bf1_, wf2, bf2_, bv_ref, o_ref, ob_ref):
        x = x_ref[...]
        hA = x + pa_ref[0] + pa_ref[1]
        hB = x + pb_ref[0] + pb_ref[1]
        fi = _mm(w1b[...], jnp.maximum(_mm(w1a[...], hA) + b1a_[...], 0.0)) \
            + b1b_[...]
        fo = _mm(w2b[...], jnp.maximum(_mm(w2a[...], hB) + b2a_[...], 0.0)) \
            + b2b_[...]
        cat = jnp.concatenate([x, fi, fo], axis=0)
        u = jnp.maximum(_mm(wf1[...], cat) + bf1_[...], 0.0)
        o = x + _mm(wf2[...], u) + bf2_[...]
        o_ref[...] = o
        ob_ref[...] = o + bv_ref[...]

    full = lambda shape: pl.BlockSpec(shape, lambda j: tuple(0 for _ in shape))
    return pl.pallas_call(
        body,
        grid=(GRID,),
        in_specs=[
            pl.BlockSpec((EMB, BN), lambda j: (0, j)),
            pl.BlockSpec((2, EMB, BN), lambda j: (0, 0, j)),
            pl.BlockSpec((2, EMB, BN), lambda j: (0, 0, j)),
            full((EMB, EMB)), full((EMB, 1)),
            full((EMB, EMB)), full((EMB, 1)),
            full((EMB, EMB)), full((EMB, 1)),
            full((EMB, EMB)), full((EMB, 1)),
            full((2 * EMB, 3 * EMB)), full((2 * EMB, 1)),
            full((EMB, 2 * EMB)), full((EMB, 1)),
            full((EMB, 1)),
        ],
        out_specs=[pl.BlockSpec((EMB, BN), lambda j: (0, j))] * 2,
        out_shape=[jax.ShapeDtypeStruct((EMB, NPAD), jnp.float32)] * 2,
    )(x_t, pA, pB, W1a, b1a, W1b, b1b, W2a, b2a, W2b, b2b,
      Wf1, bf1, Wf2, bf2, bv_col)


def _tc_final(x_t, Wc, bc_row, batch_row):
    DOUT = Wc.shape[0]

    def body(x_ref, wc_ref, bc_ref, bt_ref, o_ref, mx, sm, cn):
        j = pl.program_id(0)

        @pl.when(j == 0)
        def _():
            mx[...] = jnp.full((NUM_GRAPHS, DOUT), -jnp.inf, jnp.float32)
            sm[...] = jnp.zeros((NUM_GRAPHS, DOUT), jnp.float32)
            cn[...] = jnp.zeros((NUM_GRAPHS, 128), jnp.float32)

        yt = lax.dot_general(x_ref[...], wc_ref[...],
                             (((0,), (1,)), ((), ())),
                             preferred_element_type=jnp.float32) + bc_ref[...]
        bt = bt_ref[...]
        gids = lax.broadcasted_iota(jnp.int32, (NUM_GRAPHS, BN), 0)
        masks = (gids == bt).astype(jnp.float32)
        sm[...] += lax.dot_general(masks, yt, (((1,), (0,)), ((), ())),
                                   preferred_element_type=jnp.float32)
        cn[...] += jnp.sum(masks, axis=1, keepdims=True)
        for gph in range(NUM_GRAPHS):
            m = bt == gph
            ym = jnp.where(jnp.transpose(m), yt, -jnp.inf)
            gm = jnp.max(ym, axis=0, keepdims=True)
            mx[pl.ds(gph, 1), :] = jnp.maximum(mx[pl.ds(gph, 1), :], gm)

        @pl.when(j == GRID - 1)
        def _():
            o_ref[:, :DOUT] = mx[...]
            o_ref[:, DOUT:] = sm[...] / jnp.maximum(cn[:, :1], 1.0)

    return pl.pallas_call(
        body,
        grid=(GRID,),
        in_specs=[
            pl.BlockSpec((EMB, BN), lambda j: (0, j)),
            pl.BlockSpec((DOUT, EMB), lambda j: (0, 0)),
            pl.BlockSpec((1, DOUT), lambda j: (0, 0)),
            pl.BlockSpec((1, BN), lambda j: (0, j)),
        ],
        out_specs=pl.BlockSpec((NUM_GRAPHS, 2 * DOUT), lambda j: (0, 0)),
        out_shape=jax.ShapeDtypeStruct((NUM_GRAPHS, 2 * DOUT), jnp.float32),
        scratch_shapes=[
            pltpu.VMEM((NUM_GRAPHS, DOUT), jnp.float32),
            pltpu.VMEM((NUM_GRAPHS, DOUT), jnp.float32),
            pltpu.VMEM((NUM_GRAPHS, 128), jnp.float32),
        ],
    )(x_t, Wc, bc_row, batch_row)


def kernel(nodes, edges, edge_attr, batch,
           W_enc, b_enc, W_edge, b_edge,
           W1a, b1a, W1b, b1b, W2a, b2a, W2b, b2b,
           Wf1, bf1, Wf2, bf2, Wc, bc):
    attr_bits = lax.bitcast_convert_type(edge_attr, jnp.int32)
    epack = jnp.concatenate(
        [edges[0][None], edges[1][None], attr_bits[None]], axis=0)
    wv = W_edge[:, 0]
    bv = b_edge

    nodes_pad = jnp.pad(nodes, ((0, NPAD - N), (0, 0)))
    batch_row = jnp.pad(batch, (0, NPAD - N),
                        constant_values=NUM_GRAPHS).reshape(1, NPAD)

    bv_col = bv.reshape(EMB, 1)
    x_t, xb_t = _tc_encode(nodes_pad, W_enc, b_enc.reshape(EMB, 1), bv_col)
    for _ in range(3):
        pA, pB = _sc_aggregate(xb_t, epack, wv)
        x_t, xb_t = _tc_iter(x_t, pA, pB,
                             W1a, b1a.reshape(EMB, 1), W1b,
                             b1b.reshape(EMB, 1),
                             W2a, b2a.reshape(EMB, 1), W2b,
                             b2b.reshape(EMB, 1),
                             Wf1, bf1.reshape(2 * EMB, 1),
                             Wf2, bf2.reshape(EMB, 1), bv_col)
    return _tc_final(x_t, Wc, bc.reshape(1, 2 * EMB), batch_row)


# speedup vs baseline: 10.3282x; 1.0682x over previous
import dataclasses
import functools

import jax
import jax.numpy as jnp
from jax import lax
from jax.experimental import pallas as pl
from jax.experimental.pallas import tpu as pltpu
from jax.experimental.pallas import tpu_sc as plsc

N = 10000
NPAD = 10240
E = 320000
EMB = 64
NUM_GRAPHS = 16

C = 640
NPT = (E // 2) // C
FPT = 4
BN = 1024
GRID = NPAD // BN

_SC_MESH = plsc.VectorSubcoreMesh(core_axis_name="c", subcore_axis_name="s")

_SC_PARAMS = pltpu.CompilerParams()
if "needs_layout_passes" in pltpu.CompilerParams.__dataclass_fields__:
    _SC_PARAMS = dataclasses.replace(_SC_PARAMS, needs_layout_passes=False)


def _sc_aggregate(xb_t, epack, wv):
    out_t = [jax.ShapeDtypeStruct((2, EMB, NPAD), jnp.float32),
             jax.ShapeDtypeStruct((2, EMB, NPAD), jnp.float32)]

    row = pltpu.VMEM((NPAD,), jnp.float32)

    @functools.partial(
        pl.kernel,
        out_type=out_t,
        mesh=_SC_MESH,
        compiler_params=_SC_PARAMS,
        scratch_types=[
            [row] * FPT,
            [row] * FPT,
            [row] * FPT,
            pltpu.VMEM((3, C), jnp.int32),
            pltpu.VMEM((3, C), jnp.int32),
            pltpu.VMEM((EMB,), jnp.float32),
            pltpu.SemaphoreType.DMA,
            pltpu.SemaphoreType.DMA,
        ],
    )
    def k(x_hbm, e_hbm, w_hbm, outA, outB,
          xv, aggA, aggB, ev0, ev1, wvv, sem0, sem1):
        cid = lax.axis_index("c")
        sid = lax.axis_index("s")
        wid = cid * 16 + sid
        g = wid % 16
        half = wid // 16
        ebase = half * (E // 2)

        for f in range(FPT):
            pltpu.sync_copy(x_hbm.at[g * FPT + f], xv[f])
        pltpu.sync_copy(w_hbm, wvv)

        zero = jnp.zeros((16,), jnp.float32)

        @pl.loop(0, NPAD // 16)
        def _(i):
            sl = pl.ds(i * 16, 16)
            for f in range(FPT):
                aggA[f][sl] = zero
                aggB[f][sl] = zero

        wb = [plsc.load_gather(wvv, [jnp.full((16,), g * FPT + f, jnp.int32)])
              for f in range(FPT)]

        def issue(ci, ebuf, sem):
            pltpu.async_copy(e_hbm.at[:, pl.ds(ebase + ci * C, C)], ebuf, sem)

        def drain(ci, ebuf, sem):
            pltpu.make_async_copy(
                e_hbm.at[:, pl.ds(ebase + ci * C, C)], ebuf, sem).wait()

        def compute(ebuf):
            def loads(i):
                sl = pl.ds(i * 16, 16)
                sv = ebuf[0, sl]
                dv = ebuf[1, sl]
                av = plsc.bitcast(ebuf[2, sl], jnp.float32)
                xa = [plsc.load_gather(xv[f], [sv]) for f in range(FPT)]
                xb = [plsc.load_gather(xv[f], [dv]) for f in range(FPT)]
                mA = tuple(jnp.maximum(xa[f] + av * wb[f], 0.0)
                           for f in range(FPT))
                mB = tuple(jnp.maximum(xb[f] + av * wb[f], 0.0)
                           for f in range(FPT))
                return sv, dv, mA, mB

            def stores(sv, dv, mA, mB):
                for f in range(FPT):
                    plsc.addupdate_scatter(aggA[f], [dv], mA[f])
                for f in range(FPT):
                    plsc.addupdate_scatter(aggB[f], [sv], mB[f])

            sv0, dv0, mA0, mB0 = loads(0)

            def body(i, carry):
                psv, pdv, pmA, pmB = carry
                sv, dv, mA, mB = loads(i)
                stores(psv, pdv, pmA, pmB)
                return sv, dv, mA, mB

            carry = lax.fori_loop(1, C // 16, body,
                                  (sv0, dv0, mA0, mB0), unroll=2)
            stores(*carry)

        issue(0, ev0, sem0)

        @pl.loop(0, NPT, step=2)
        def _(ci):
            drain(ci, ev0, sem0)
            issue(ci + 1, ev1, sem1)
            compute(ev0)
            drain(ci + 1, ev1, sem1)

            @pl.when(ci + 2 < NPT)
            def _():
                issue(ci + 2, ev0, sem0)

            compute(ev1)

        for f in range(FPT):
            pltpu.sync_copy(aggA[f], outA.at[half, g * FPT + f])
            pltpu.sync_copy(aggB[f], outB.at[half, g * FPT + f])

    return k(xb_t, epack, wv)


def _mm(a, b):
    return lax.dot_general(a, b, (((1,), (0,)), ((), ())),
                           preferred_element_type=jnp.float32)


def _tc_encode(nodes_pad, W_enc, b_enc_col, bv_col):
    d_in = nodes_pad.shape[1]

    def body(n_ref, w_ref, b_ref, bv_ref, o_ref, ob_ref):
        o = lax.dot_general(
            w_ref[...], n_ref[...], (((1,), (1,)), ((), ())),
            preferred_element_type=jnp.float32) + b_ref[...]
        o_ref[...] = o
        ob_ref[...] = o + bv_ref[...]

    return pl.pallas_call(
        body,
        grid=(GRID,),
        in_specs=[
            pl.BlockSpec((BN, d_in), lambda j: (j, 0)),
            pl.BlockSpec((EMB, d_in), lambda j: (0, 0)),
            pl.BlockSpec((EMB, 1), lambda j: (0, 0)),
            pl.BlockSpec((EMB, 1), lambda j: (0, 0)),
        ],
        out_specs=[pl.BlockSpec((EMB, BN), lambda j: (0, j))] * 2,
        out_shape=[jax.ShapeDtypeStruct((EMB, NPAD), jnp.float32)] * 2,
    )(nodes_pad, W_enc, b_enc_col, bv_col)


def _tc_iter(x_t, pA, pB, W1a, b1a, W1b, b1b, W2a, b2a, W2b, b2b,
             Wf1, bf1, Wf2, bf2, bv_col):
    def body(x_ref, pa_ref, pb_ref,
             w1a, b1a_, w1b, b1b_, w2a, b2a_, w2b, b2b_,
             wf1, bf1_, wf2, bf2_, bv_ref, o_ref, ob_ref):
        x = x_ref[...]
        hA = x + pa_ref[0] + pa_ref[1]
        hB = x + pb_ref[0] + pb_ref[1]
        fi = _mm(w1b[...], jnp.maximum(_mm(w1a[...], hA) + b1a_[...], 0.0)) \
            + b1b_[...]
        fo = _mm(w2b[...], jnp.maximum(_mm(w2a[...], hB) + b2a_[...], 0.0)) \
            + b2b_[...]
        cat = jnp.concatenate([x, fi, fo], axis=0)
        u = jnp.maximum(_mm(wf1[...], cat) + bf1_[...], 0.0)
        o = x + _mm(wf2[...], u) + bf2_[...]
        o_ref[...] = o
        ob_ref[...] = o + bv_ref[...]

    full = lambda shape: pl.BlockSpec(shape, lambda j: tuple(0 for _ in shape))
    return pl.pallas_call(
        body,
        grid=(GRID,),
        in_specs=[
            pl.BlockSpec((EMB, BN), lambda j: (0, j)),
            pl.BlockSpec((2, EMB, BN), lambda j: (0, 0, j)),
            pl.BlockSpec((2, EMB, BN), lambda j: (0, 0, j)),
            full((EMB, EMB)), full((EMB, 1)),
            full((EMB, EMB)), full((EMB, 1)),
            full((EMB, EMB)), full((EMB, 1)),
            full((EMB, EMB)), full((EMB, 1)),
            full((2 * EMB, 3 * EMB)), full((2 * EMB, 1)),
            full((EMB, 2 * EMB)), full((EMB, 1)),
            full((EMB, 1)),
        ],
        out_specs=[pl.BlockSpec((EMB, BN), lambda j: (0, j))] * 2,
        out_shape=[jax.ShapeDtypeStruct((EMB, NPAD), jnp.float32)] * 2,
    )(x_t, pA, pB, W1a, b1a, W1b, b1b, W2a, b2a, W2b, b2b,
      Wf1, bf1, Wf2, bf2, bv_col)


def _tc_final(x_t, Wc, bc_row, batch_row):
    DOUT = Wc.shape[0]

    def body(x_ref, wc_ref, bc_ref, bt_ref, o_ref, mx, sm, cn):
        j = pl.program_id(0)

        @pl.when(j == 0)
        def _():
            mx[...] = jnp.full((NUM_GRAPHS, DOUT), -jnp.inf, jnp.float32)
            sm[...] = jnp.zeros((NUM_GRAPHS, DOUT), jnp.float32)
            cn[...] = jnp.zeros((NUM_GRAPHS, 128), jnp.float32)

        yt = lax.dot_general(x_ref[...], wc_ref[...],
                             (((0,), (1,)), ((), ())),
                             preferred_element_type=jnp.float32) + bc_ref[...]
        bt = bt_ref[...]
        gids = lax.broadcasted_iota(jnp.int32, (NUM_GRAPHS, BN), 0)
        masks = (gids == bt).astype(jnp.float32)
        sm[...] += lax.dot_general(masks, yt, (((1,), (0,)), ((), ())),
                                   preferred_element_type=jnp.float32)
        cn[...] += jnp.sum(masks, axis=1, keepdims=True)
        for gph in range(NUM_GRAPHS):
            m = bt == gph
            ym = jnp.where(jnp.transpose(m), yt, -jnp.inf)
            gm = jnp.max(ym, axis=0, keepdims=True)
            mx[pl.ds(gph, 1), :] = jnp.maximum(mx[pl.ds(gph, 1), :], gm)

        @pl.when(j == GRID - 1)
        def _():
            o_ref[:, :DOUT] = mx[...]
            o_ref[:, DOUT:] = sm[...] / jnp.maximum(cn[:, :1], 1.0)

    return pl.pallas_call(
        body,
        grid=(GRID,),
        in_specs=[
            pl.BlockSpec((EMB, BN), lambda j: (0, j)),
            pl.BlockSpec((DOUT, EMB), lambda j: (0, 0)),
            pl.BlockSpec((1, DOUT), lambda j: (0, 0)),
            pl.BlockSpec((1, BN), lambda j: (0, j)),
        ],
        out_specs=pl.BlockSpec((NUM_GRAPHS, 2 * DOUT), lambda j: (0, 0)),
        out_shape=jax.ShapeDtypeStruct((NUM_GRAPHS, 2 * DOUT), jnp.float32),
        scratch_shapes=[
            pltpu.VMEM((NUM_GRAPHS, DOUT), jnp.float32),
            pltpu.VMEM((NUM_GRAPHS, DOUT), jnp.float32),
            pltpu.VMEM((NUM_GRAPHS, 128), jnp.float32),
        ],
    )(x_t, Wc, bc_row, batch_row)


def kernel(nodes, edges, edge_attr, batch,
           W_enc, b_enc, W_edge, b_edge,
           W1a, b1a, W1b, b1b, W2a, b2a, W2b, b2b,
           Wf1, bf1, Wf2, bf2, Wc, bc):
    attr_bits = lax.bitcast_convert_type(edge_attr, jnp.int32)
    epack = jnp.concatenate(
        [edges[0][None], edges[1][None], attr_bits[None]], axis=0)
    wv = W_edge[:, 0]
    bv = b_edge

    nodes_pad = jnp.pad(nodes, ((0, NPAD - N), (0, 0)))
    batch_row = jnp.pad(batch, (0, NPAD - N),
                        constant_values=NUM_GRAPHS).reshape(1, NPAD)

    bv_col = bv.reshape(EMB, 1)
    x_t, xb_t = _tc_encode(nodes_pad, W_enc, b_enc.reshape(EMB, 1), bv_col)
    for _ in range(3):
        pA, pB = _sc_aggregate(xb_t, epack, wv)
        x_t, xb_t = _tc_iter(x_t, pA, pB,
                             W1a, b1a.reshape(EMB, 1), W1b,
                             b1b.reshape(EMB, 1),
                             W2a, b2a.reshape(EMB, 1), W2b,
                             b2b.reshape(EMB, 1),
                             Wf1, bf1.reshape(2 * EMB, 1),
                             Wf2, bf2.reshape(EMB, 1), bv_col)
    return _tc_final(x_t, Wc, bc.reshape(1, 2 * EMB), batch_row)


# speedup vs baseline: 11.9465x; 1.1567x over previous
import dataclasses
import functools

import jax
import jax.numpy as jnp
from jax import lax
from jax.experimental import pallas as pl
from jax.experimental.pallas import tpu as pltpu
from jax.experimental.pallas import tpu_sc as plsc

N = 10000
NPAD = 10240
E = 320000
EMB = 64
NUM_GRAPHS = 16

C = 640
NPT = (E // 2) // C
FPT = 4
BN = 1024
GRID = NPAD // BN

_SC_MESH = plsc.VectorSubcoreMesh(core_axis_name="c", subcore_axis_name="s")

_SC_PARAMS = pltpu.CompilerParams()
if "needs_layout_passes" in pltpu.CompilerParams.__dataclass_fields__:
    _SC_PARAMS = dataclasses.replace(_SC_PARAMS, needs_layout_passes=False)


def _sc_aggregate(xp_t, epack, wv):
    out_t = [jax.ShapeDtypeStruct((2, EMB, NPAD), jnp.float32),
             jax.ShapeDtypeStruct((2, EMB, NPAD), jnp.float32)]

    row = pltpu.VMEM((NPAD,), jnp.float32)
    rowi = pltpu.VMEM((NPAD,), jnp.int32)

    @functools.partial(
        pl.kernel,
        out_type=out_t,
        mesh=_SC_MESH,
        compiler_params=_SC_PARAMS,
        scratch_types=[
            [rowi] * 2,
            [row] * FPT,
            [row] * FPT,
            pltpu.VMEM((3, C), jnp.int32),
            pltpu.VMEM((3, C), jnp.int32),
            pltpu.VMEM((EMB,), jnp.float32),
            pltpu.SemaphoreType.DMA,
            pltpu.SemaphoreType.DMA,
        ],
    )
    def k(x_hbm, e_hbm, w_hbm, outA, outB,
          xp, aggA, aggB, ev0, ev1, wvv, sem0, sem1):
        cid = lax.axis_index("c")
        sid = lax.axis_index("s")
        wid = cid * 16 + sid
        g = wid % 16
        half = wid // 16
        ebase = half * (E // 2)
        feats = [2 * g, 2 * g + 1, 2 * g + 32, 2 * g + 33]

        for r in range(2):
            pltpu.sync_copy(x_hbm.at[2 * g + r], xp[r])
        pltpu.sync_copy(w_hbm, wvv)

        zero = jnp.zeros((16,), jnp.float32)

        @pl.loop(0, NPAD // 16)
        def _(i):
            sl = pl.ds(i * 16, 16)
            for f in range(FPT):
                aggA[f][sl] = zero
                aggB[f][sl] = zero

        wb = [plsc.load_gather(wvv, [feats[f] + jnp.zeros((16,), jnp.int32)])
              for f in range(FPT)]

        def unpack(p):
            lo = plsc.bitcast(p << 16, jnp.float32)
            hi = plsc.bitcast(p & jnp.int32(-65536), jnp.float32)
            return lo, hi

        def issue(ci, ebuf, sem):
            pltpu.async_copy(e_hbm.at[:, pl.ds(ebase + ci * C, C)], ebuf, sem)

        def drain(ci, ebuf, sem):
            pltpu.make_async_copy(
                e_hbm.at[:, pl.ds(ebase + ci * C, C)], ebuf, sem).wait()

        def compute(ebuf):
            def loads(i):
                sl = pl.ds(i * 16, 16)
                sv = ebuf[0, sl]
                dv = ebuf[1, sl]
                av = plsc.bitcast(ebuf[2, sl], jnp.float32)
                pa = [plsc.load_gather(xp[r], [sv]) for r in range(2)]
                pb = [plsc.load_gather(xp[r], [dv]) for r in range(2)]
                xa0, xa2 = unpack(pa[0])
                xa1, xa3 = unpack(pa[1])
                xb0, xb2 = unpack(pb[0])
                xb1, xb3 = unpack(pb[1])
                xa = [xa0, xa1, xa2, xa3]
                xb = [xb0, xb1, xb2, xb3]
                mA = tuple(jnp.maximum(xa[f] + av * wb[f], 0.0)
                           for f in range(FPT))
                mB = tuple(jnp.maximum(xb[f] + av * wb[f], 0.0)
                           for f in range(FPT))
                return sv, dv, mA, mB

            def stores(sv, dv, mA, mB):
                for f in range(FPT):
                    plsc.addupdate_scatter(aggA[f], [dv], mA[f])
                for f in range(FPT):
                    plsc.addupdate_scatter(aggB[f], [sv], mB[f])

            sv0, dv0, mA0, mB0 = loads(0)

            def body(i, carry):
                psv, pdv, pmA, pmB = carry
                sv, dv, mA, mB = loads(i)
                stores(psv, pdv, pmA, pmB)
                return sv, dv, mA, mB

            carry = lax.fori_loop(1, C // 16, body,
                                  (sv0, dv0, mA0, mB0), unroll=2)
            stores(*carry)

        issue(0, ev0, sem0)

        @pl.loop(0, NPT, step=2)
        def _(ci):
            drain(ci, ev0, sem0)
            issue(ci + 1, ev1, sem1)
            compute(ev0)
            drain(ci + 1, ev1, sem1)

            @pl.when(ci + 2 < NPT)
            def _():
                issue(ci + 2, ev0, sem0)

            compute(ev1)

        for f in range(FPT):
            pltpu.sync_copy(aggA[f], outA.at[half, feats[f]])
            pltpu.sync_copy(aggB[f], outB.at[half, feats[f]])

    return k(xp_t, epack, wv)


def _mm(a, b):
    return lax.dot_general(a, b, (((1,), (0,)), ((), ())),
                           preferred_element_type=jnp.float32)


def _pack_rows(xb):
    rnd = lax.bitcast_convert_type(xb, jnp.int32) + jnp.int32(0x8000)
    top = lax.shift_right_logical(rnd[:EMB // 2, :], 16)
    bot = rnd[EMB // 2:, :] & jnp.int32(-65536)
    return bot | top


def _tc_encode(nodes_pad, W_enc, b_enc_col, bv_col):
    d_in = nodes_pad.shape[1]

    def body(n_ref, w_ref, b_ref, bv_ref, o_ref, op_ref):
        o = lax.dot_general(
            w_ref[...], n_ref[...], (((1,), (1,)), ((), ())),
            preferred_element_type=jnp.float32) + b_ref[...]
        o_ref[...] = o
        op_ref[...] = _pack_rows(o + bv_ref[...])

    return pl.pallas_call(
        body,
        grid=(GRID,),
        in_specs=[
            pl.BlockSpec((BN, d_in), lambda j: (j, 0)),
            pl.BlockSpec((EMB, d_in), lambda j: (0, 0)),
            pl.BlockSpec((EMB, 1), lambda j: (0, 0)),
            pl.BlockSpec((EMB, 1), lambda j: (0, 0)),
        ],
        out_specs=[pl.BlockSpec((EMB, BN), lambda j: (0, j)),
                   pl.BlockSpec((EMB // 2, BN), lambda j: (0, j))],
        out_shape=[jax.ShapeDtypeStruct((EMB, NPAD), jnp.float32),
                   jax.ShapeDtypeStruct((EMB // 2, NPAD), jnp.int32)],
    )(nodes_pad, W_enc, b_enc_col, bv_col)


def _tc_iter(x_t, pA, pB, W1a, b1a, W1b, b1b, W2a, b2a, W2b, b2b,
             Wf1, bf1, Wf2, bf2, bv_col):
    def body(x_ref, pa_ref, pb_ref,
             w1a, b1a_, w1b, b1b_, w2a, b2a_, w2b, b2b_,
             wf1, bf1_, wf2, bf2_, bv_ref, o_ref, op_ref):
        x = x_ref[...]
        hA = x + pa_ref[0] + pa_ref[1]
        hB = x + pb_ref[0] + pb_ref[1]
        fi = _mm(w1b[...], jnp.maximum(_mm(w1a[...], hA) + b1a_[...], 0.0)) \
            + b1b_[...]
        fo = _mm(w2b[...], jnp.maximum(_mm(w2a[...], hB) + b2a_[...], 0.0)) \
            + b2b_[...]
        cat = jnp.concatenate([x, fi, fo], axis=0)
        u = jnp.maximum(_mm(wf1[...], cat) + bf1_[...], 0.0)
        o = x + _mm(wf2[...], u) + bf2_[...]
        o_ref[...] = o
        op_ref[...] = _pack_rows(o + bv_ref[...])

    full = lambda shape: pl.BlockSpec(shape, lambda j: tuple(0 for _ in shape))
    return pl.pallas_call(
        body,
        grid=(GRID,),
        in_specs=[
            pl.BlockSpec((EMB, BN), lambda j: (0, j)),
            pl.BlockSpec((2, EMB, BN), lambda j: (0, 0, j)),
            pl.BlockSpec((2, EMB, BN), lambda j: (0, 0, j)),
            full((EMB, EMB)), full((EMB, 1)),
            full((EMB, EMB)), full((EMB, 1)),
            full((EMB, EMB)), full((EMB, 1)),
            full((EMB, EMB)), full((EMB, 1)),
            full((2 * EMB, 3 * EMB)), full((2 * EMB, 1)),
            full((EMB, 2 * EMB)), full((EMB, 1)),
            full((EMB, 1)),
        ],
        out_specs=[pl.BlockSpec((EMB, BN), lambda j: (0, j)),
                   pl.BlockSpec((EMB // 2, BN), lambda j: (0, j))],
        out_shape=[jax.ShapeDtypeStruct((EMB, NPAD), jnp.float32),
                   jax.ShapeDtypeStruct((EMB // 2, NPAD), jnp.int32)],
    )(x_t, pA, pB, W1a, b1a, W1b, b1b, W2a, b2a, W2b, b2b,
      Wf1, bf1, Wf2, bf2, bv_col)


def _tc_final(x_t, Wc, bc_row, batch_row):
    DOUT = Wc.shape[0]

    def body(x_ref, wc_ref, bc_ref, bt_ref, o_ref, mx, sm, cn):
        j = pl.program_id(0)

        @pl.when(j == 0)
        def _():
            mx[...] = jnp.full((NUM_GRAPHS, DOUT), -jnp.inf, jnp.float32)
            sm[...] = jnp.zeros((NUM_GRAPHS, DOUT), jnp.float32)
            cn[...] = jnp.zeros((NUM_GRAPHS, 128), jnp.float32)

        yt = lax.dot_general(x_ref[...], wc_ref[...],
                             (((0,), (1,)), ((), ())),
                             preferred_element_type=jnp.float32) + bc_ref[...]
        bt = bt_ref[...]
        gids = lax.broadcasted_iota(jnp.int32, (NUM_GRAPHS, BN), 0)
        masks = (gids == bt).astype(jnp.float32)
        sm[...] += lax.dot_general(masks, yt, (((1,), (0,)), ((), ())),
                                   preferred_element_type=jnp.float32)
        cn[...] += jnp.sum(masks, axis=1, keepdims=True)
        for gph in range(NUM_GRAPHS):
            m = bt == gph
            ym = jnp.where(jnp.transpose(m), yt, -jnp.inf)
            gm = jnp.max(ym, axis=0, keepdims=True)
            mx[pl.ds(gph, 1), :] = jnp.maximum(mx[pl.ds(gph, 1), :], gm)

        @pl.when(j == GRID - 1)
        def _():
            o_ref[:, :DOUT] = mx[...]
            o_ref[:, DOUT:] = sm[...] / jnp.maximum(cn[:, :1], 1.0)

    return pl.pallas_call(
        body,
        grid=(GRID,),
        in_specs=[
            pl.BlockSpec((EMB, BN), lambda j: (0, j)),
            pl.BlockSpec((DOUT, EMB), lambda j: (0, 0)),
            pl.BlockSpec((1, DOUT), lambda j: (0, 0)),
            pl.BlockSpec((1, BN), lambda j: (0, j)),
        ],
        out_specs=pl.BlockSpec((NUM_GRAPHS, 2 * DOUT), lambda j: (0, 0)),
        out_shape=jax.ShapeDtypeStruct((NUM_GRAPHS, 2 * DOUT), jnp.float32),
        scratch_shapes=[
            pltpu.VMEM((NUM_GRAPHS, DOUT), jnp.float32),
            pltpu.VMEM((NUM_GRAPHS, DOUT), jnp.float32),
            pltpu.VMEM((NUM_GRAPHS, 128), jnp.float32),
        ],
    )(x_t, Wc, bc_row, batch_row)


def kernel(nodes, edges, edge_attr, batch,
           W_enc, b_enc, W_edge, b_edge,
           W1a, b1a, W1b, b1b, W2a, b2a, W2b, b2b,
           Wf1, bf1, Wf2, bf2, Wc, bc):
    attr_bits = lax.bitcast_convert_type(edge_attr, jnp.int32)
    epack = jnp.concatenate(
        [edges[0][None], edges[1][None], attr_bits[None]], axis=0)
    wv = W_edge[:, 0]
    bv = b_edge

    nodes_pad = jnp.pad(nodes, ((0, NPAD - N), (0, 0)))
    batch_row = jnp.pad(batch, (0, NPAD - N),
                        constant_values=NUM_GRAPHS).reshape(1, NPAD)

    bv_col = bv.reshape(EMB, 1)
    x_t, xp_t = _tc_encode(nodes_pad, W_enc, b_enc.reshape(EMB, 1), bv_col)
    for _ in range(3):
        pA, pB = _sc_aggregate(xp_t, epack, wv)
        x_t, xp_t = _tc_iter(x_t, pA, pB,
                             W1a, b1a.reshape(EMB, 1), W1b,
                             b1b.reshape(EMB, 1),
                             W2a, b2a.reshape(EMB, 1), W2b,
                             b2b.reshape(EMB, 1),
                             Wf1, bf1.reshape(2 * EMB, 1),
                             Wf2, bf2.reshape(EMB, 1), bv_col)
    return _tc_final(x_t, Wc, bc.reshape(1, 2 * EMB), batch_row)
